# baseline (device time: 127642 ns/iter reference)
import jax
import jax.numpy as jnp
from jax import lax
from jax.experimental import pallas as pl
from jax.experimental.pallas import tpu as pltpu

B = 2
S = 1024
S_HALF = S // 2
N = 2048

NC = 8
CHUNK = B * S_HALF // NC


def _chunk_bs(c):
    per_b = NC // B
    return c // per_b, (c % per_b) * CHUNK


def kernel(O, Wo):
    b, s, h, d = O.shape

    def body(o_ref, wo_ref, out_ref, send_buf, recv_buf, send_sems, recv_sems):
        my_x = lax.axis_index("x")
        my_y = lax.axis_index("y")
        peer = (1 - my_x, my_y)

        barrier_sem = pltpu.get_barrier_semaphore()
        pl.semaphore_signal(
            barrier_sem, inc=1,
            device_id=peer, device_id_type=pl.DeviceIdType.MESH,
        )
        pl.semaphore_wait(barrier_sem, 1)

        wo = wo_ref[:, :]
        peer_off = (1 - my_x) * S_HALF
        my_off = my_x * S_HALF

        rdmas = []
        for c in range(NC):
            bb, off = _chunk_bs(c)
            o_chunk = jnp.reshape(
                o_ref[bb, pl.ds(peer_off + off, CHUNK), :, :], (CHUNK, 1024)
            )
            send_buf[bb, pl.ds(off, CHUNK), :] = jnp.dot(
                o_chunk, wo, preferred_element_type=jnp.float32,
            )
            rdma = pltpu.make_async_remote_copy(
                src_ref=send_buf.at[bb, pl.ds(off, CHUNK), :],
                dst_ref=recv_buf.at[bb, pl.ds(off, CHUNK), :],
                send_sem=send_sems.at[c],
                recv_sem=recv_sems.at[c],
                device_id=peer,
                device_id_type=pl.DeviceIdType.MESH,
            )
            rdma.start()
            rdmas.append(rdma)

        for c in range(NC):
            bb, off = _chunk_bs(c)
            o_mine = jnp.reshape(
                o_ref[bb, pl.ds(my_off + off, CHUNK), :, :], (CHUNK, 1024)
            )
            out_ref[bb, pl.ds(off, CHUNK), :] = jnp.dot(
                o_mine, wo, preferred_element_type=jnp.float32,
            )

        for c in range(NC):
            bb, off = _chunk_bs(c)
            rdmas[c].wait_recv()
            out_ref[bb, pl.ds(off, CHUNK), :] += recv_buf[bb, pl.ds(off, CHUNK), :]
        for c in range(NC):
            rdmas[c].wait_send()

    return pl.pallas_call(
        body,
        out_shape=jax.ShapeDtypeStruct((B, S_HALF, N), jnp.float32),
        in_specs=[
            pl.BlockSpec(memory_space=pltpu.VMEM),
            pl.BlockSpec(memory_space=pltpu.VMEM),
        ],
        out_specs=pl.BlockSpec(memory_space=pltpu.VMEM),
        scratch_shapes=[
            pltpu.VMEM((B, S_HALF, N), jnp.float32),
            pltpu.VMEM((B, S_HALF, N), jnp.float32),
            pltpu.SemaphoreType.DMA((NC,)),
            pltpu.SemaphoreType.DMA((NC,)),
        ],
        compiler_params=pltpu.CompilerParams(
            collective_id=0, vmem_limit_bytes=56 * 1024 * 1024
        ),
    )(O, Wo)


# device time: 61041 ns/iter; 2.0911x vs baseline; 2.0911x over previous
import jax
import jax.numpy as jnp
from jax import lax
from jax.experimental import pallas as pl
from jax.experimental.pallas import tpu as pltpu

B = 2
S = 1024
S_HALF = S // 2
N = 2048

NC = 8
CHUNK = B * S_HALF // NC


def _chunk_bs(c):
    per_b = NC // B
    return c // per_b, (c % per_b) * CHUNK


def kernel(O, Wo):
    b, s, h, d = O.shape
    o_flat = O.reshape(b, s, h * d).astype(jnp.bfloat16)
    wo_b = Wo.astype(jnp.bfloat16)

    def body(o_ref, wo_ref, out_ref, send_buf, recv_buf, send_sems, recv_sems):
        my_x = lax.axis_index("x")
        my_y = lax.axis_index("y")
        peer = (1 - my_x, my_y)

        barrier_sem = pltpu.get_barrier_semaphore()
        pl.semaphore_signal(
            barrier_sem, inc=1,
            device_id=peer, device_id_type=pl.DeviceIdType.MESH,
        )
        pl.semaphore_wait(barrier_sem, 1)

        wo = wo_ref[:, :]
        peer_off = (1 - my_x) * S_HALF
        my_off = my_x * S_HALF

        rdmas = []
        for c in range(NC):
            bb, off = _chunk_bs(c)
            send_buf[bb, pl.ds(off, CHUNK), :] = jnp.dot(
                o_ref[bb, pl.ds(peer_off + off, CHUNK), :], wo,
                preferred_element_type=jnp.float32,
            ).astype(jnp.bfloat16)
            rdma = pltpu.make_async_remote_copy(
                src_ref=send_buf.at[bb, pl.ds(off, CHUNK), :],
                dst_ref=recv_buf.at[bb, pl.ds(off, CHUNK), :],
                send_sem=send_sems.at[c],
                recv_sem=recv_sems.at[c],
                device_id=peer,
                device_id_type=pl.DeviceIdType.MESH,
            )
            rdma.start()
            rdmas.append(rdma)

        for bb in range(B):
            out_ref[bb, :, :] = jnp.dot(
                o_ref[bb, pl.ds(my_off, S_HALF), :], wo,
                preferred_element_type=jnp.float32,
            )

        for c in range(NC):
            bb, off = _chunk_bs(c)
            rdmas[c].wait_recv()
            out_ref[bb, pl.ds(off, CHUNK), :] += recv_buf[
                bb, pl.ds(off, CHUNK), :
            ].astype(jnp.float32)
        for c in range(NC):
            rdmas[c].wait_send()

    return pl.pallas_call(
        body,
        out_shape=jax.ShapeDtypeStruct((B, S_HALF, N), jnp.float32),
        in_specs=[
            pl.BlockSpec(memory_space=pltpu.VMEM),
            pl.BlockSpec(memory_space=pltpu.VMEM),
        ],
        out_specs=pl.BlockSpec(memory_space=pltpu.VMEM),
        scratch_shapes=[
            pltpu.VMEM((B, S_HALF, N), jnp.bfloat16),
            pltpu.VMEM((B, S_HALF, N), jnp.bfloat16),
            pltpu.SemaphoreType.DMA((NC,)),
            pltpu.SemaphoreType.DMA((NC,)),
        ],
        compiler_params=pltpu.CompilerParams(collective_id=0),
    )(o_flat, wo_b)


# device time: 59766 ns/iter; 2.1357x vs baseline; 1.0213x over previous
import jax
import jax.numpy as jnp
from jax import lax
from jax.experimental import pallas as pl
from jax.experimental.pallas import tpu as pltpu

B = 2
S = 1024
S_HALF = S // 2
N = 2048

NC = 8
CHUNK = B * S_HALF // NC


def _chunk_bs(c):
    per_b = NC // B
    return c // per_b, (c % per_b) * CHUNK


def kernel(O, Wo):
    b, s, h, d = O.shape
    o_t = jnp.swapaxes(O.reshape(b, s, h * d), 1, 2).astype(jnp.bfloat16)
    wo_b = Wo.astype(jnp.bfloat16)

    def body(o_ref, wo_ref, out_ref, send_buf, recv_buf, send_sems, recv_sems):
        my_x = lax.axis_index("x")
        my_y = lax.axis_index("y")
        peer = (1 - my_x, my_y)

        barrier_sem = pltpu.get_barrier_semaphore()
        pl.semaphore_signal(
            barrier_sem, inc=1,
            device_id=peer, device_id_type=pl.DeviceIdType.MESH,
        )
        pl.semaphore_wait(barrier_sem, 1)

        wo = wo_ref[:, :]
        peer_off = (1 - my_x) * S_HALF
        my_off = my_x * S_HALF

        rdmas = []
        for c in range(NC):
            bb, off = _chunk_bs(c)
            send_buf[bb, pl.ds(off, CHUNK), :] = lax.dot_general(
                o_ref[bb, :, pl.ds(peer_off + off, CHUNK)], wo,
                (((0,), (0,)), ((), ())),
                preferred_element_type=jnp.float32,
            ).astype(jnp.bfloat16)
            rdma = pltpu.make_async_remote_copy(
                src_ref=send_buf.at[bb, pl.ds(off, CHUNK), :],
                dst_ref=recv_buf.at[bb, pl.ds(off, CHUNK), :],
                send_sem=send_sems.at[c],
                recv_sem=recv_sems.at[c],
                device_id=peer,
                device_id_type=pl.DeviceIdType.MESH,
            )
            rdma.start()
            rdmas.append(rdma)

        for c in range(NC):
            bb, off = _chunk_bs(c)
            out_ref[bb, pl.ds(off, CHUNK), :] = lax.dot_general(
                o_ref[bb, :, pl.ds(my_off + off, CHUNK)], wo,
                (((0,), (0,)), ((), ())),
                preferred_element_type=jnp.float32,
            )

        for c in range(NC):
            bb, off = _chunk_bs(c)
            rdmas[c].wait_recv()
            out_ref[bb, pl.ds(off, CHUNK), :] += recv_buf[
                bb, pl.ds(off, CHUNK), :
            ].astype(jnp.float32)
        for c in range(NC):
            rdmas[c].wait_send()

    return pl.pallas_call(
        body,
        out_shape=jax.ShapeDtypeStruct((B, S_HALF, N), jnp.float32),
        in_specs=[
            pl.BlockSpec(memory_space=pltpu.VMEM),
            pl.BlockSpec(memory_space=pltpu.VMEM),
        ],
        out_specs=pl.BlockSpec(memory_space=pltpu.VMEM),
        scratch_shapes=[
            pltpu.VMEM((B, S_HALF, N), jnp.bfloat16),
            pltpu.VMEM((B, S_HALF, N), jnp.bfloat16),
            pltpu.SemaphoreType.DMA((NC,)),
            pltpu.SemaphoreType.DMA((NC,)),
        ],
        compiler_params=pltpu.CompilerParams(collective_id=0),
    )(o_t, wo_b)
